# baseline (speedup 1.0000x reference)
import functools

import jax
import jax.numpy as jnp
from jax import lax
from jax.experimental import pallas as pl
from jax.experimental.pallas import tpu as pltpu
from jax.experimental.pallas import tpu_sc as plsc

PAD = 1
B = 4
S = 4096
D = 1024

NC = 2
NS = 16
L = 16
NW = NC * NS
TOK_PER_W = (B * S) // NW
TILES_PER_BATCH = S // TOK_PER_W
NGROUP = TOK_PER_W // L
K = 32
NCHUNK = TOK_PER_W // K
KGROUP = K // L

_mesh = plsc.VectorSubcoreMesh(core_axis_name="c", subcore_axis_name="s")


@functools.partial(
    pl.kernel,
    out_type=jax.ShapeDtypeStruct((B * S, D), jnp.float32),
    mesh=_mesh,
    compiler_params=pltpu.CompilerParams(needs_layout_passes=False),
    scratch_types=[
        pltpu.VMEM((TOK_PER_W,), jnp.int32),
        pltpu.VMEM((NCHUNK, K), jnp.int32),
        pltpu.VMEM((L,), jnp.int32),
        pltpu.VMEM((NS, L), jnp.int32),
        pltpu.VMEM_SHARED((NS, L), jnp.int32),
        pltpu.VMEM((K, D), jnp.float32),
        pltpu.VMEM((K, D), jnp.float32),
        pltpu.SemaphoreType.DMA,
        pltpu.SemaphoreType.DMA,
    ],
)
def _emb_kernel(inp_hbm, table_hbm, out_hbm,
                tok_v, idx_v, stage_v, totals_v, totals_sh,
                rows0_v, rows1_v, gsem, wsem):
    cid = lax.axis_index("c")
    sid = lax.axis_index("s")
    w = cid * NS + sid
    base = w * TOK_PER_W

    pltpu.sync_copy(inp_hbm.at[pl.ds(base, TOK_PER_W)], tok_v)
    pref = jnp.int32(0)
    for i in range(NGROUP):
        tok = tok_v[pl.ds(i * L, L)]
        m = jnp.minimum(jnp.abs(tok - PAD), 1)
        c = plsc.cumsum(m)
        pos = (pref + c) * m + PAD
        idx_v[i // KGROUP, pl.ds((i % KGROUP) * L, L)] = pos
        pref = pref + jnp.max(c)

    stage_v[...] = jnp.full((L,), pref, jnp.int32)
    pltpu.sync_copy(stage_v, totals_sh.at[sid])
    plsc.subcore_barrier()
    pltpu.sync_copy(totals_sh, totals_v)
    lb = sid // TILES_PER_BATCH
    acc = jnp.zeros((L,), jnp.int32)
    for p in range(NS):
        same_row = jnp.int32(1) - jnp.minimum(
            jnp.abs(jnp.int32(p // TILES_PER_BATCH) - lb), 1)
        before = jnp.minimum(jnp.maximum(sid - jnp.int32(p), 0), 1)
        acc = acc + totals_v[p, :] * (same_row * before)
    gpref = jnp.max(acc)

    for r in range(NCHUNK):
        for o in range(KGROUP):
            v = idx_v[r, pl.ds(o * L, L)]
            v = v + gpref * jnp.minimum(v - PAD, 1)
            idx_v[r, pl.ds(o * L, L)] = v

    bufs = (rows0_v, rows1_v)
    gco = [None] * NCHUNK
    wco = [None] * NCHUNK
    gco[0] = pltpu.async_copy(table_hbm.at[idx_v.at[0]], bufs[0], gsem)
    for c in range(NCHUNK):
        cur = bufs[c % 2]
        gco[c].wait()
        if c + 1 < NCHUNK:
            if c >= 1:
                wco[c - 1].wait()
            gco[c + 1] = pltpu.async_copy(
                table_hbm.at[idx_v.at[c + 1]], bufs[(c + 1) % 2], gsem)
        wco[c] = pltpu.async_copy(cur, out_hbm.at[pl.ds(base + c * K, K)], wsem)
    wco[NCHUNK - 1].wait()


def kernel(input, table):
    out = _emb_kernel(input.reshape(-1), table)
    return out.reshape(B, S, D)


# speedup vs baseline: 1.0048x; 1.0048x over previous
import functools

import jax
import jax.numpy as jnp
from jax import lax
from jax.experimental import pallas as pl
from jax.experimental.pallas import tpu as pltpu
from jax.experimental.pallas import tpu_sc as plsc

PAD = 1
B = 4
S = 4096
D = 1024

NC = 2
NS = 16
L = 16
NW = NC * NS
TOK_PER_W = (B * S) // NW
TILES_PER_BATCH = S // TOK_PER_W
NGROUP = TOK_PER_W // L
K = 32
NCHUNK = TOK_PER_W // K
KGROUP = K // L

_mesh = plsc.VectorSubcoreMesh(core_axis_name="c", subcore_axis_name="s")


@functools.partial(
    pl.kernel,
    out_type=jax.ShapeDtypeStruct((B * S, D), jnp.float32),
    mesh=_mesh,
    compiler_params=pltpu.CompilerParams(needs_layout_passes=False),
    scratch_types=[
        pltpu.VMEM((TOK_PER_W,), jnp.int32),
        pltpu.VMEM((NCHUNK, K), jnp.int32),
        pltpu.VMEM((L,), jnp.int32),
        pltpu.VMEM((NS, L), jnp.int32),
        pltpu.VMEM_SHARED((NS, L), jnp.int32),
        pltpu.VMEM((K, D), jnp.float32),
        pltpu.VMEM((K, D), jnp.float32),
        pltpu.VMEM((K, D), jnp.float32),
        pltpu.SemaphoreType.DMA,
        pltpu.SemaphoreType.DMA,
    ],
)
def _emb_kernel(inp_hbm, table_hbm, out_hbm,
                tok_v, idx_v, stage_v, totals_v, totals_sh,
                rows0_v, rows1_v, rows2_v, gsem, wsem):
    cid = lax.axis_index("c")
    sid = lax.axis_index("s")
    w = cid * NS + sid
    base = w * TOK_PER_W

    pltpu.sync_copy(inp_hbm.at[pl.ds(base, TOK_PER_W)], tok_v)
    pref = jnp.int32(0)
    for i in range(NGROUP):
        tok = tok_v[pl.ds(i * L, L)]
        m = jnp.minimum(jnp.abs(tok - PAD), 1)
        c = plsc.cumsum(m)
        pos = (pref + c) * m + PAD
        idx_v[i // KGROUP, pl.ds((i % KGROUP) * L, L)] = pos
        pref = pref + jnp.max(c)

    stage_v[...] = jnp.full((L,), pref, jnp.int32)
    pltpu.sync_copy(stage_v, totals_sh.at[sid])
    plsc.subcore_barrier()
    pltpu.sync_copy(totals_sh, totals_v)
    lb = sid // TILES_PER_BATCH
    acc = jnp.zeros((L,), jnp.int32)
    for p in range(NS):
        same_row = jnp.int32(1) - jnp.minimum(
            jnp.abs(jnp.int32(p // TILES_PER_BATCH) - lb), 1)
        before = jnp.minimum(jnp.maximum(sid - jnp.int32(p), 0), 1)
        acc = acc + totals_v[p, :] * (same_row * before)
    gpref = jnp.max(acc)

    for r in range(NCHUNK):
        for o in range(KGROUP):
            v = idx_v[r, pl.ds(o * L, L)]
            v = v + gpref * jnp.minimum(v - PAD, 1)
            idx_v[r, pl.ds(o * L, L)] = v

    bufs = (rows0_v, rows1_v, rows2_v)
    gco = [None] * NCHUNK
    wco = [None] * NCHUNK
    gco[0] = pltpu.async_copy(table_hbm.at[idx_v.at[0]], bufs[0], gsem)
    for c in range(NCHUNK):
        cur = bufs[c % 3]
        gco[c].wait()
        if c + 1 < NCHUNK:
            if c >= 2:
                wco[c - 2].wait()
            gco[c + 1] = pltpu.async_copy(
                table_hbm.at[idx_v.at[c + 1]], bufs[(c + 1) % 3], gsem)
        wco[c] = pltpu.async_copy(cur, out_hbm.at[pl.ds(base + c * K, K)], wsem)
    wco[NCHUNK - 2].wait()
    wco[NCHUNK - 1].wait()


def kernel(input, table):
    out = _emb_kernel(input.reshape(-1), table)
    return out.reshape(B, S, D)
